# baseline (speedup 1.0000x reference)
import jax
import jax.numpy as jnp
from jax.experimental import pallas as pl

N = 10000
E = 320000
D = 128
K = N // 4


def _mm_kernel(x_ref, m_ref, ws_ref, wn_ref, b_ref, o_ref):
    o_ref[...] = (
        jnp.dot(x_ref[...], ws_ref[...], preferred_element_type=jnp.float32)
        + jnp.dot(m_ref[...], wn_ref[...], preferred_element_type=jnp.float32)
        + b_ref[...]
    )


def _sage_dense(x, mean, w_self, w_neigh, b):
    BN = 1000
    return pl.pallas_call(
        _mm_kernel,
        grid=(N // BN,),
        in_specs=[
            pl.BlockSpec((BN, D), lambda i: (i, 0)),
            pl.BlockSpec((BN, D), lambda i: (i, 0)),
            pl.BlockSpec((D, D), lambda i: (0, 0)),
            pl.BlockSpec((D, D), lambda i: (0, 0)),
            pl.BlockSpec((1, D), lambda i: (0, 0)),
        ],
        out_specs=pl.BlockSpec((BN, D), lambda i: (i, 0)),
        out_shape=jax.ShapeDtypeStruct((N, D), jnp.float32),
    )(x, mean, w_self, w_neigh, b.reshape(1, D))


def kernel(x, edge_index, w_self1, w_neigh1, b1, w_self2, w_neigh2, b2, w_score, b_score):
    src = edge_index[0]
    dst = edge_index[1]
    ones = jnp.ones((E, 1), dtype=jnp.float32)
    cnt = jax.ops.segment_sum(ones, dst, num_segments=N)
    inv = 1.0 / jnp.maximum(cnt, 1.0)

    agg1 = jax.ops.segment_sum(x[src], dst, num_segments=N)
    h = _sage_dense(x, agg1 * inv, w_self1, w_neigh1, b1)
    h = jax.nn.relu(h)
    agg2 = jax.ops.segment_sum(h[src], dst, num_segments=N)
    h = _sage_dense(h, agg2 * inv, w_self2, w_neigh2, b2)

    score = jax.nn.sigmoid(h @ w_score + b_score)
    _, idx = jax.lax.top_k(score[:, 0], K)
    compressed = h[idx, :]
    return compressed, idx


# speedup vs baseline: 1.4661x; 1.4661x over previous
import functools

import jax
import jax.numpy as jnp
from jax import lax
from jax.experimental import pallas as pl
from jax.experimental.pallas import tpu as pltpu
from jax.experimental.pallas import tpu_sc as plsc

N = 10000
E = 320000
D = 128
K = N // 4

NC = 2
NS = 16
NW = NC * NS
EPW = E // NW
GB = 80
NB = EPW // GB


def _gather_sc(table, src):
    mesh = plsc.VectorSubcoreMesh(core_axis_name="c", subcore_axis_name="s")

    @functools.partial(
        pl.kernel,
        mesh=mesh,
        out_type=jax.ShapeDtypeStruct((E, D), jnp.float32),
        scratch_types=[
            pltpu.VMEM((EPW,), jnp.int32),
            pltpu.VMEM((GB, D), jnp.float32),
            pltpu.SemaphoreType.DMA,
        ],
    )
    def k(table_hbm, src_hbm, out_hbm, idx_v, rows_v, sem):
        wid = lax.axis_index("s") * NC + lax.axis_index("c")
        base = wid * EPW
        pltpu.sync_copy(src_hbm.at[pl.ds(base, EPW)], idx_v)

        def body(b, carry):
            off = b * GB
            pltpu.async_copy(
                table_hbm.at[idx_v.at[pl.ds(off, GB)]], rows_v, sem
            ).wait()
            pltpu.sync_copy(rows_v, out_hbm.at[pl.ds(base + off, GB)])
            return carry

        lax.fori_loop(0, NB, body, 0)

    return k(table, src)


def _mm_kernel(x_ref, m_ref, ws_ref, wn_ref, b_ref, o_ref):
    o_ref[...] = (
        jnp.dot(x_ref[...], ws_ref[...], preferred_element_type=jnp.float32)
        + jnp.dot(m_ref[...], wn_ref[...], preferred_element_type=jnp.float32)
        + b_ref[...]
    )


def _sage_dense(x, mean, w_self, w_neigh, b):
    BN = 1000
    return pl.pallas_call(
        _mm_kernel,
        grid=(N // BN,),
        in_specs=[
            pl.BlockSpec((BN, D), lambda i: (i, 0)),
            pl.BlockSpec((BN, D), lambda i: (i, 0)),
            pl.BlockSpec((D, D), lambda i: (0, 0)),
            pl.BlockSpec((D, D), lambda i: (0, 0)),
            pl.BlockSpec((1, D), lambda i: (0, 0)),
        ],
        out_specs=pl.BlockSpec((BN, D), lambda i: (i, 0)),
        out_shape=jax.ShapeDtypeStruct((N, D), jnp.float32),
    )(x, mean, w_self, w_neigh, b.reshape(1, D))


def kernel(x, edge_index, w_self1, w_neigh1, b1, w_self2, w_neigh2, b2, w_score, b_score):
    src = edge_index[0]
    dst = edge_index[1]
    ones = jnp.ones((E, 1), dtype=jnp.float32)
    cnt = jax.ops.segment_sum(ones, dst, num_segments=N)
    inv = 1.0 / jnp.maximum(cnt, 1.0)

    msg1 = _gather_sc(x, src)
    agg1 = jax.ops.segment_sum(msg1, dst, num_segments=N)
    h = _sage_dense(x, agg1 * inv, w_self1, w_neigh1, b1)
    h = jax.nn.relu(h)
    msg2 = _gather_sc(h, src)
    agg2 = jax.ops.segment_sum(msg2, dst, num_segments=N)
    h = _sage_dense(h, agg2 * inv, w_self2, w_neigh2, b2)

    score = jax.nn.sigmoid(h @ w_score + b_score)
    _, idx = jax.lax.top_k(score[:, 0], K)
    compressed = h[idx, :]
    return compressed, idx


# speedup vs baseline: 1.5154x; 1.0336x over previous
import functools

import jax
import jax.numpy as jnp
from jax import lax
from jax.experimental import pallas as pl
from jax.experimental.pallas import tpu as pltpu
from jax.experimental.pallas import tpu_sc as plsc

N = 10000
E = 320000
D = 128
K = N // 4

NC = 2
NS = 16
NW = NC * NS
EPW = E // NW
GB = 80
NB = EPW // GB


def _gather_sc(table, src):
    mesh = plsc.VectorSubcoreMesh(core_axis_name="c", subcore_axis_name="s")

    @functools.partial(
        pl.kernel,
        mesh=mesh,
        out_type=jax.ShapeDtypeStruct((E, D), jnp.float32),
        scratch_types=[
            pltpu.VMEM((EPW,), jnp.int32),
            pltpu.VMEM((GB, D), jnp.float32),
            pltpu.VMEM((GB, D), jnp.float32),
            pltpu.SemaphoreType.DMA,
            pltpu.SemaphoreType.DMA,
        ],
    )
    def k(table_hbm, src_hbm, out_hbm, idx_v, rows_v0, rows_v1, sem0, sem1):
        wid = lax.axis_index("s") * NC + lax.axis_index("c")
        base = wid * EPW
        pltpu.sync_copy(src_hbm.at[pl.ds(base, EPW)], idx_v)
        bufs = (rows_v0, rows_v1)
        sems = (sem0, sem1)

        def issue(b):
            return pltpu.async_copy(
                table_hbm.at[idx_v.at[pl.ds(b * GB, GB)]],
                bufs[b % 2], sems[b % 2]
            )

        h = issue(0)
        for b in range(NB):
            h_next = issue(b + 1) if b + 1 < NB else None
            h.wait()
            pltpu.sync_copy(bufs[b % 2], out_hbm.at[pl.ds(base + b * GB, GB)])
            h = h_next

    return k(table, src)


def _mm_kernel(x_ref, m_ref, ws_ref, wn_ref, b_ref, o_ref):
    o_ref[...] = (
        jnp.dot(x_ref[...], ws_ref[...], preferred_element_type=jnp.float32)
        + jnp.dot(m_ref[...], wn_ref[...], preferred_element_type=jnp.float32)
        + b_ref[...]
    )


def _sage_dense(x, mean, w_self, w_neigh, b):
    BN = 1000
    return pl.pallas_call(
        _mm_kernel,
        grid=(N // BN,),
        in_specs=[
            pl.BlockSpec((BN, D), lambda i: (i, 0)),
            pl.BlockSpec((BN, D), lambda i: (i, 0)),
            pl.BlockSpec((D, D), lambda i: (0, 0)),
            pl.BlockSpec((D, D), lambda i: (0, 0)),
            pl.BlockSpec((1, D), lambda i: (0, 0)),
        ],
        out_specs=pl.BlockSpec((BN, D), lambda i: (i, 0)),
        out_shape=jax.ShapeDtypeStruct((N, D), jnp.float32),
    )(x, mean, w_self, w_neigh, b.reshape(1, D))


def kernel(x, edge_index, w_self1, w_neigh1, b1, w_self2, w_neigh2, b2, w_score, b_score):
    src = edge_index[0]
    dst = edge_index[1]
    ones = jnp.ones((E, 1), dtype=jnp.float32)
    cnt = jax.ops.segment_sum(ones, dst, num_segments=N)
    inv = 1.0 / jnp.maximum(cnt, 1.0)

    msg1 = _gather_sc(x, src)
    agg1 = jax.ops.segment_sum(msg1, dst, num_segments=N)
    h = _sage_dense(x, agg1 * inv, w_self1, w_neigh1, b1)
    h = jax.nn.relu(h)
    msg2 = _gather_sc(h, src)
    agg2 = jax.ops.segment_sum(msg2, dst, num_segments=N)
    h = _sage_dense(h, agg2 * inv, w_self2, w_neigh2, b2)

    score = jax.nn.sigmoid(h @ w_score + b_score)
    _, idx = jax.lax.top_k(score[:, 0], K)
    compressed = h[idx, :]
    return compressed, idx
